# baseline (speedup 1.0000x reference)
import jax
import jax.numpy as jnp
import numpy as np
from jax import lax
from jax.experimental import pallas as pl
from jax.experimental.pallas import tpu as pltpu
from jax.experimental.pallas import tpu_sc as plsc

_EPS = np.float32(np.finfo(np.float64).eps)

_NC, _NS, _LANES = 2, 16, 16
_NW = _NC * _NS
_L_SC = 256
_CHUNK = 8
_TC_TILE = 448


def _tc_combine(g_ref, x_ref, o_ref):
    b = pl.program_id(0)
    num_e = x_ref.shape[0]
    acc = jnp.exp(x_ref[0, 0]) * g_ref[b, 0]
    for e in range(1, num_e):
        acc = acc + jnp.exp(x_ref[e, 0]) * g_ref[b, e]
    acc = jnp.where(acc == 0.0, _EPS, acc)
    o_ref[0] = jnp.log(acc)


def _tc_log(c_ref, full_ref, o_ref):
    del full_ref
    acc = c_ref[0]
    acc = jnp.where(acc == 0.0, _EPS, acc)
    o_ref[0] = jnp.log(acc)


def _sc_combine(xs_ref, g_ref, out_ref, xbuf, gbuf, obuf, sem):
    num_e, num_b, seq_len, dim = xs_ref.shape
    l_base = seq_len - _L_SC
    chunks_per_b = _L_SC // _CHUNK
    tasks_per_w = (num_b * chunks_per_b) // _NW
    wid = lax.axis_index("s") * _NC + lax.axis_index("c")
    for t in range(tasks_per_w):
        task = wid * tasks_per_w + t
        b = task // chunks_per_b
        lc = (task % chunks_per_b) * _CHUNK
        pltpu.sync_copy(g_ref.at[b], gbuf)
        copies = [
            pltpu.async_copy(
                xs_ref.at[e, b, pl.ds(l_base + lc, _CHUNK), :], xbuf.at[e], sem
            )
            for e in range(num_e)
        ]
        for c in copies:
            c.wait()

        def row_body(r, carry):
            for j in range(dim // _LANES):
                sl = pl.ds(j * _LANES, _LANES)
                acc = jnp.exp(xbuf[0, r, sl]) * gbuf[0, :]
                for e in range(1, num_e):
                    acc = acc + jnp.exp(xbuf[e, r, sl]) * gbuf[e, :]
                obuf[r, sl] = acc
            return carry

        lax.fori_loop(0, _CHUNK, row_body, 0)
        pltpu.sync_copy(obuf, out_ref.at[b, pl.ds(lc, _CHUNK), :])


def kernel(xs, gates):
    num_b, num_e = gates.shape
    _, seq_len, dim = xs.shape
    l_tc = seq_len - _L_SC
    xs4 = xs.reshape(num_e, num_b, seq_len, dim)
    garr = jnp.broadcast_to(gates[:, :, None], (num_b, num_e, _LANES))

    tc_out = pl.pallas_call(
        _tc_combine,
        grid=(num_b, l_tc // _TC_TILE),
        in_specs=[
            pl.BlockSpec(memory_space=pltpu.SMEM),
            pl.BlockSpec((num_e, 1, _TC_TILE, dim), lambda b, l: (0, b, l, 0)),
        ],
        out_specs=pl.BlockSpec((1, _TC_TILE, dim), lambda b, l: (b, l, 0)),
        out_shape=jax.ShapeDtypeStruct((num_b, seq_len, dim), jnp.float32),
        compiler_params=pltpu.CompilerParams(
            dimension_semantics=("parallel", "parallel")
        ),
    )(gates, xs4)

    mesh = plsc.VectorSubcoreMesh(
        core_axis_name="c", subcore_axis_name="s", num_cores=_NC,
        num_subcores=_NS,
    )
    sc_combined = pl.kernel(
        _sc_combine,
        out_type=jax.ShapeDtypeStruct((num_b, _L_SC, dim), jnp.float32),
        mesh=mesh,
        scratch_types=[
            pltpu.VMEM((num_e, _CHUNK, dim), jnp.float32),
            pltpu.VMEM((num_e, _LANES), jnp.float32),
            pltpu.VMEM((_CHUNK, dim), jnp.float32),
            pltpu.SemaphoreType.DMA,
        ],
    )(xs4, garr)

    return pl.pallas_call(
        _tc_log,
        grid=(num_b, 1),
        in_specs=[
            pl.BlockSpec((1, _L_SC, dim), lambda b, j: (b, j, 0)),
            pl.BlockSpec(memory_space=pl.ANY),
        ],
        out_specs=pl.BlockSpec(
            (1, _L_SC, dim), lambda b, j: (b, l_tc // _L_SC + j, 0)
        ),
        out_shape=jax.ShapeDtypeStruct((num_b, seq_len, dim), jnp.float32),
        input_output_aliases={1: 0},
    )(sc_combined, tc_out)


# speedup vs baseline: 1.2020x; 1.2020x over previous
import jax
import jax.numpy as jnp
import numpy as np
from jax import lax
from jax.experimental import pallas as pl
from jax.experimental.pallas import tpu as pltpu
from jax.experimental.pallas import tpu_sc as plsc

_EPS = np.float32(np.finfo(np.float64).eps)

_NC, _NS, _LANES = 2, 16, 16
_NW = _NC * _NS
_L_SC = 256
_CHUNK = 8
_TC_TILE = 448


def _tc_combine(g_ref, x_ref, o_ref):
    b = pl.program_id(0)
    num_e = x_ref.shape[0]
    acc = jnp.exp(x_ref[0, 0]) * g_ref[b, 0]
    for e in range(1, num_e):
        acc = acc + jnp.exp(x_ref[e, 0]) * g_ref[b, e]
    acc = jnp.where(acc == 0.0, _EPS, acc)
    o_ref[0] = jnp.log(acc)


def _tc_log(c_ref, full_ref, o_ref):
    del full_ref
    acc = c_ref[0]
    acc = jnp.where(acc == 0.0, _EPS, acc)
    o_ref[0] = jnp.log(acc)


def _sc_combine(xs_ref, g_ref, out_ref, xbuf, gbuf, obuf, sems):
    num_e, num_b, seq_len, dim = xs_ref.shape
    l_base = seq_len - _L_SC
    chunks_per_b = _L_SC // _CHUNK
    tasks_per_w = (num_b * chunks_per_b) // _NW
    wid = lax.axis_index("s") * _NC + lax.axis_index("c")

    def start_fetch(slot, t):
        task = wid * tasks_per_w + t
        b = task // chunks_per_b
        lc = (task % chunks_per_b) * _CHUNK
        return [
            pltpu.async_copy(
                xs_ref.at[e, b, pl.ds(l_base + lc, _CHUNK), :],
                xbuf.at[slot, e],
                sems.at[slot],
            )
            for e in range(num_e)
        ]

    copies = start_fetch(0, 0)
    for t in range(tasks_per_w):
        slot = t % 2
        task = wid * tasks_per_w + t
        b = task // chunks_per_b
        lc = (task % chunks_per_b) * _CHUNK
        pltpu.sync_copy(g_ref.at[b], gbuf)
        next_copies = (
            start_fetch(1 - slot, t + 1) if t + 1 < tasks_per_w else ()
        )
        for c in copies:
            c.wait()

        def row_body(r, carry):
            for j in range(dim // _LANES):
                sl = pl.ds(j * _LANES, _LANES)
                acc = jnp.exp(xbuf[slot, 0, r, sl]) * gbuf[0, :]
                for e in range(1, num_e):
                    acc = acc + jnp.exp(xbuf[slot, e, r, sl]) * gbuf[e, :]
                obuf[r, sl] = acc
            return carry

        lax.fori_loop(0, _CHUNK, row_body, 0)
        pltpu.sync_copy(obuf, out_ref.at[b, pl.ds(lc, _CHUNK), :])
        copies = next_copies


def kernel(xs, gates):
    num_b, num_e = gates.shape
    _, seq_len, dim = xs.shape
    l_tc = seq_len - _L_SC
    xs4 = xs.reshape(num_e, num_b, seq_len, dim)
    garr = jnp.broadcast_to(gates[:, :, None], (num_b, num_e, _LANES))

    tc_out = pl.pallas_call(
        _tc_combine,
        grid=(num_b, l_tc // _TC_TILE),
        in_specs=[
            pl.BlockSpec(memory_space=pltpu.SMEM),
            pl.BlockSpec((num_e, 1, _TC_TILE, dim), lambda b, l: (0, b, l, 0)),
        ],
        out_specs=pl.BlockSpec((1, _TC_TILE, dim), lambda b, l: (b, l, 0)),
        out_shape=jax.ShapeDtypeStruct((num_b, seq_len, dim), jnp.float32),
        compiler_params=pltpu.CompilerParams(
            dimension_semantics=("parallel", "parallel")
        ),
    )(gates, xs4)

    mesh = plsc.VectorSubcoreMesh(
        core_axis_name="c", subcore_axis_name="s", num_cores=_NC,
        num_subcores=_NS,
    )
    sc_combined = pl.kernel(
        _sc_combine,
        out_type=jax.ShapeDtypeStruct((num_b, _L_SC, dim), jnp.float32),
        mesh=mesh,
        scratch_types=[
            pltpu.VMEM((2, num_e, _CHUNK, dim), jnp.float32),
            pltpu.VMEM((num_e, _LANES), jnp.float32),
            pltpu.VMEM((_CHUNK, dim), jnp.float32),
            pltpu.SemaphoreType.DMA((2,)),
        ],
    )(xs4, garr)

    return pl.pallas_call(
        _tc_log,
        grid=(num_b, 1),
        in_specs=[
            pl.BlockSpec((1, _L_SC, dim), lambda b, j: (b, j, 0)),
            pl.BlockSpec(memory_space=pl.ANY),
        ],
        out_specs=pl.BlockSpec(
            (1, _L_SC, dim), lambda b, j: (b, l_tc // _L_SC + j, 0)
        ),
        out_shape=jax.ShapeDtypeStruct((num_b, seq_len, dim), jnp.float32),
        input_output_aliases={1: 0},
    )(sc_combined, tc_out)


# speedup vs baseline: 1.2023x; 1.0003x over previous
import jax
import jax.numpy as jnp
import numpy as np
from jax import lax
from jax.experimental import pallas as pl
from jax.experimental.pallas import tpu as pltpu
from jax.experimental.pallas import tpu_sc as plsc

_EPS = np.float32(np.finfo(np.float64).eps)

_NC, _NS, _LANES = 2, 16, 16
_NW = _NC * _NS
_L_SC = 256
_CHUNK = 8
_TC_TILE = 448


def _tc_combine(g_ref, x_ref, o_ref):
    b = pl.program_id(0)
    num_e = x_ref.shape[0]
    acc = jnp.exp(x_ref[0, 0]) * g_ref[b, 0]
    for e in range(1, num_e):
        acc = acc + jnp.exp(x_ref[e, 0]) * g_ref[b, e]
    acc = jnp.where(acc == 0.0, _EPS, acc)
    o_ref[0] = jnp.log(acc)


def _tc_log(c_ref, full_ref, o_ref):
    del full_ref
    acc = c_ref[0]
    acc = jnp.where(acc == 0.0, _EPS, acc)
    o_ref[0] = jnp.log(acc)


def _sc_combine(xs_ref, g_ref, out_ref, xbuf, gbuf, obuf, sems):
    num_e, num_b, seq_len, dim = xs_ref.shape
    l_base = seq_len - _L_SC
    chunks_per_b = _L_SC // _CHUNK
    tasks_per_w = (num_b * chunks_per_b) // _NW
    wid = lax.axis_index("s") * _NC + lax.axis_index("c")

    def start_fetch(slot, t):
        task = wid * tasks_per_w + t
        b = task // chunks_per_b
        lc = (task % chunks_per_b) * _CHUNK
        return [
            pltpu.async_copy(
                xs_ref.at[e, b, pl.ds(l_base + lc, _CHUNK), :],
                xbuf.at[slot, e],
                sems.at[slot],
            )
            for e in range(num_e)
        ]

    copies = start_fetch(0, 0)
    for t in range(tasks_per_w):
        slot = t % 2
        task = wid * tasks_per_w + t
        b = task // chunks_per_b
        lc = (task % chunks_per_b) * _CHUNK
        pltpu.sync_copy(g_ref.at[b], gbuf)
        next_copies = (
            start_fetch(1 - slot, t + 1) if t + 1 < tasks_per_w else ()
        )
        for c in copies:
            c.wait()

        def row_body(r, carry):
            for j in range(dim // _LANES):
                sl = pl.ds(j * _LANES, _LANES)
                acc = jnp.exp(xbuf[slot, 0, r, sl]) * gbuf[0, :]
                for e in range(1, num_e):
                    acc = acc + jnp.exp(xbuf[slot, e, r, sl]) * gbuf[e, :]
                obuf[r, sl] = acc
            return carry

        lax.fori_loop(0, _CHUNK, row_body, 0)
        pltpu.sync_copy(obuf, out_ref.at[b, pl.ds(lc, _CHUNK), :])
        copies = next_copies


def kernel(xs, gates):
    num_b, num_e = gates.shape
    _, seq_len, dim = xs.shape
    l_tc = seq_len - _L_SC
    xs4 = xs.reshape(num_e, num_b, seq_len, dim)
    garr = jnp.broadcast_to(gates[:, :, None], (num_b, num_e, _LANES))

    mesh = plsc.VectorSubcoreMesh(
        core_axis_name="c", subcore_axis_name="s", num_cores=_NC,
        num_subcores=_NS,
    )
    sc_combined = pl.kernel(
        _sc_combine,
        out_type=jax.ShapeDtypeStruct((num_b, _L_SC, dim), jnp.float32),
        mesh=mesh,
        scratch_types=[
            pltpu.VMEM((2, num_e, _CHUNK, dim), jnp.float32),
            pltpu.VMEM((num_e, _LANES), jnp.float32),
            pltpu.VMEM((_CHUNK, dim), jnp.float32),
            pltpu.SemaphoreType.DMA((2,)),
        ],
    )(xs4, garr)

    tc_out = pl.pallas_call(
        _tc_combine,
        grid=(num_b, l_tc // _TC_TILE),
        in_specs=[
            pl.BlockSpec(memory_space=pltpu.SMEM),
            pl.BlockSpec((num_e, 1, _TC_TILE, dim), lambda b, l: (0, b, l, 0)),
        ],
        out_specs=pl.BlockSpec((1, _TC_TILE, dim), lambda b, l: (b, l, 0)),
        out_shape=jax.ShapeDtypeStruct((num_b, seq_len, dim), jnp.float32),
        compiler_params=pltpu.CompilerParams(
            dimension_semantics=("parallel", "parallel")
        ),
    )(gates, xs4)

    return pl.pallas_call(
        _tc_log,
        grid=(num_b, 1),
        in_specs=[
            pl.BlockSpec((1, _L_SC, dim), lambda b, j: (b, j, 0)),
            pl.BlockSpec(memory_space=pl.ANY),
        ],
        out_specs=pl.BlockSpec(
            (1, _L_SC, dim), lambda b, j: (b, l_tc // _L_SC + j, 0)
        ),
        out_shape=jax.ShapeDtypeStruct((num_b, seq_len, dim), jnp.float32),
        input_output_aliases={1: 0},
    )(sc_combined, tc_out)


# speedup vs baseline: 1.5836x; 1.3172x over previous
import jax
import jax.numpy as jnp
import numpy as np
from jax.experimental import pallas as pl
from jax.experimental.pallas import tpu as pltpu

_EPS = np.float32(np.finfo(np.float64).eps)


def _combine_kernel(g_ref, x_ref, o_ref):
    b = pl.program_id(0)
    num_e = x_ref.shape[0]
    acc = jnp.exp(x_ref[0, 0]) * g_ref[b, 0]
    for e in range(1, num_e):
        acc = acc + jnp.exp(x_ref[e, 0]) * g_ref[b, e]
    acc = jnp.where(acc == 0.0, _EPS, acc)
    o_ref[0] = jnp.log(acc)


def kernel(xs, gates):
    num_b, num_e = gates.shape
    _, seq_len, dim = xs.shape
    xs4 = xs.reshape(num_e, num_b, seq_len, dim)
    tile = 512
    return pl.pallas_call(
        _combine_kernel,
        grid=(num_b, seq_len // tile),
        in_specs=[
            pl.BlockSpec(memory_space=pltpu.SMEM),
            pl.BlockSpec((num_e, 1, tile, dim), lambda b, l: (0, b, l, 0)),
        ],
        out_specs=pl.BlockSpec((1, tile, dim), lambda b, l: (b, l, 0)),
        out_shape=jax.ShapeDtypeStruct((num_b, seq_len, dim), jnp.float32),
        compiler_params=pltpu.CompilerParams(
            dimension_semantics=("parallel", "parallel")
        ),
    )(gates, xs4)
